# baseline (speedup 1.0000x reference)
import functools

import jax
import jax.numpy as jnp
from jax import lax
from jax.experimental import pallas as pl
from jax.experimental.pallas import tpu as pltpu
from jax.experimental.pallas import tpu_sc as plsc

_NUM_CORES = 2
_NUM_SUBCORES = 16
_NUM_WORKERS = _NUM_CORES * _NUM_SUBCORES
_CHUNK = 64


def kernel(x, embedding):
    seq_len = x.shape[1]
    features = embedding.shape[1]
    rows_per_worker = seq_len // _NUM_WORKERS
    n_chunks = rows_per_worker // _CHUNK
    mesh = plsc.VectorSubcoreMesh(core_axis_name="c", subcore_axis_name="s")

    @functools.partial(
        pl.kernel,
        out_type=jax.ShapeDtypeStruct((seq_len, features), embedding.dtype),
        mesh=mesh,
        scratch_types=[pltpu.VMEM((_CHUNK, features), jnp.float32)],
    )
    def sc_copy(emb_hbm, out_hbm, buf):
        wid = lax.axis_index("s") * _NUM_CORES + lax.axis_index("c")
        base = wid * rows_per_worker
        for j in range(n_chunks):
            off = base + j * _CHUNK
            pltpu.sync_copy(emb_hbm.at[pl.ds(off, _CHUNK)], buf)
            pltpu.sync_copy(buf, out_hbm.at[pl.ds(off, _CHUNK)])

    return sc_copy(embedding)


# speedup vs baseline: 1.0040x; 1.0040x over previous
import functools

import jax
import jax.numpy as jnp
from jax import lax
from jax.experimental import pallas as pl
from jax.experimental.pallas import tpu as pltpu
from jax.experimental.pallas import tpu_sc as plsc

_NUM_CORES = 2
_NUM_SUBCORES = 16
_NUM_WORKERS = _NUM_CORES * _NUM_SUBCORES
_CHUNK = 32
_NBUF = 3


def kernel(x, embedding):
    seq_len = x.shape[1]
    features = embedding.shape[1]
    rows_per_worker = seq_len // _NUM_WORKERS
    n_chunks = rows_per_worker // _CHUNK
    mesh = plsc.VectorSubcoreMesh(core_axis_name="c", subcore_axis_name="s")

    @functools.partial(
        pl.kernel,
        out_type=jax.ShapeDtypeStruct((seq_len, features), embedding.dtype),
        mesh=mesh,
        scratch_types=[
            pltpu.VMEM((_NBUF, _CHUNK, features), jnp.float32),
            pltpu.SemaphoreType.DMA((_NBUF,)),
            pltpu.SemaphoreType.DMA((_NBUF,)),
        ],
    )
    def sc_copy(emb_hbm, out_hbm, buf, in_sems, out_sems):
        wid = lax.axis_index("s") * _NUM_CORES + lax.axis_index("c")
        base = wid * rows_per_worker

        def in_copy(j):
            b = j % _NBUF
            return pltpu.make_async_copy(
                emb_hbm.at[pl.ds(base + j * _CHUNK, _CHUNK)],
                buf.at[b],
                in_sems.at[b],
            )

        def out_copy(j):
            b = j % _NBUF
            return pltpu.make_async_copy(
                buf.at[b],
                out_hbm.at[pl.ds(base + j * _CHUNK, _CHUNK)],
                out_sems.at[b],
            )

        in_copy(0).start()
        in_copy(1).start()
        waited_out = set()
        for j in range(n_chunks):
            in_copy(j).wait()
            out_copy(j).start()
            if j + 2 < n_chunks:
                if j >= 1:
                    out_copy(j - 1).wait()
                    waited_out.add(j - 1)
                in_copy(j + 2).start()
        for j in range(n_chunks):
            if j not in waited_out:
                out_copy(j).wait()

    return sc_copy(embedding)
